# baseline (speedup 1.0000x reference)
import functools

import jax
import jax.numpy as jnp
from jax import lax
from jax.experimental import pallas as pl
from jax.experimental.pallas import tpu as pltpu
from jax.experimental.pallas import tpu_sc as plsc

N = 10000
D = 128
L_SEQ = 12
E = 320000

NC = 2
NS = 16
NW = NC * NS
LANES = 16

C = 128
EPW0 = 20480
EPW1 = 0
NCH0 = EPW0 // C
NCH1 = EPW1 // C
E_PAD = NS * (EPW0 + EPW1)
EPW_D = E_PAD // NW
CH = 2048
ACC_N = 10240
ROWS_PER_SUB = ACC_N // NS

RBLK = 1000
GRID = N // RBLK


def _mesh():
    return plsc.VectorSubcoreMesh(core_axis_name="c", subcore_axis_name="s")


def _sc_deg(colp, ewp):
    @functools.partial(
        pl.kernel,
        mesh=_mesh(),
        out_type=jax.ShapeDtypeStruct((NW, N), jnp.float32),
        compiler_params=pltpu.CompilerParams(needs_layout_passes=False),
        scratch_types=[
            pltpu.VMEM((N,), jnp.float32),
            pltpu.VMEM((CH,), jnp.int32),
            pltpu.VMEM((CH,), jnp.float32),
        ],
    )
    def deg_kernel(colp_hbm, ewp_hbm, out_hbm, hist, cbuf, ebuf):
        cid = lax.axis_index("c")
        sid = lax.axis_index("s")
        wid = sid * NC + cid

        def zero_body(i, carry):
            hist[pl.ds(i * LANES, LANES)] = jnp.zeros((LANES,), jnp.float32)
            return carry

        lax.fori_loop(0, N // LANES, zero_body, 0)

        def chunk_body(g, carry):
            base = pl.multiple_of(wid * EPW_D + g * CH, 8)
            pltpu.sync_copy(colp_hbm.at[pl.ds(base, CH)], cbuf)
            pltpu.sync_copy(ewp_hbm.at[pl.ds(base, CH)], ebuf)

            def inner(k, carry2):
                sl = pl.ds(k * LANES, LANES)
                plsc.addupdate_scatter(hist, [cbuf[sl]], ebuf[sl])
                return carry2

            lax.fori_loop(0, CH // LANES, inner, 0)
            return carry

        lax.fori_loop(0, EPW_D // CH, chunk_body, 0)
        pltpu.sync_copy(hist, out_hbm.at[wid])

    return deg_kernel(colp, ewp)


def _sc_agg(hp, rowp, colp, ewp, zeros_nd):
    @functools.partial(
        pl.kernel,
        mesh=_mesh(),
        out_type=jax.ShapeDtypeStruct((NC, ACC_N, D), jnp.float32),
        scratch_types=[
            pltpu.VMEM_SHARED((ACC_N, D), jnp.float32),
            pltpu.VMEM((2, C), jnp.int32),
            pltpu.VMEM((2, C), jnp.int32),
            pltpu.VMEM((2, C), jnp.float32),
            pltpu.VMEM((2, C, D), jnp.float32),
            pltpu.SemaphoreType.DMA((2,)),
            pltpu.SemaphoreType.DMA((2,)),
        ],
    )
    def agg_kernel(hp_hbm, rowp_hbm, colp_hbm, ewp_hbm, z_hbm, out_hbm,
                   acc, rbuf, cbuf, ebuf, rows, isem, gsem):
        cid = lax.axis_index("c")
        sid = lax.axis_index("s")
        r0 = sid * ROWS_PER_SUB
        ebase = lax.select(cid == 0, sid * EPW0, NS * EPW0 + sid * EPW1)
        npair = lax.select(cid == 0, NCH0 // 2, NCH1 // 2)

        @pl.when(cid == 0)
        def _():
            pltpu.sync_copy(z_hbm.at[pl.ds(r0, ROWS_PER_SUB)],
                            acc.at[pl.ds(r0, ROWS_PER_SUB)])
        plsc.subcore_barrier()

        def idx_copies(g, b):
            base = pl.multiple_of(ebase + g * C, 8)
            return (
                pltpu.make_async_copy(rowp_hbm.at[pl.ds(base, C)],
                                      rbuf.at[b], isem.at[b]),
                pltpu.make_async_copy(colp_hbm.at[pl.ds(base, C)],
                                      cbuf.at[b], isem.at[b]),
                pltpu.make_async_copy(ewp_hbm.at[pl.ds(base, C)],
                                      ebuf.at[b], isem.at[b]),
            )

        def issue_idx(g, b):
            for d in idx_copies(g, b):
                d.start()

        def wait_idx(g, b):
            for d in idx_copies(g, b):
                d.wait()

        def issue_gather(b):
            pltpu.async_copy(hp_hbm.at[rbuf.at[b]], rows.at[b], gsem.at[b])

        def wait_gather(b):
            pltpu.make_async_copy(hp_hbm.at[rbuf.at[b]], rows.at[b],
                                  gsem.at[b]).wait()

        def process(g, b):
            wait_gather(b)

            def scale_row(k, carry2):
                ev = ebuf[b, pl.ds(k * LANES, LANES)]
                for i in range(LANES):
                    r = k * LANES + i
                    s = ev[i]
                    for j in range(D // LANES):
                        sl = pl.ds(j * LANES, LANES)
                        rows[b, r, sl] = rows[b, r, sl] * s
                return carry2

            lax.fori_loop(0, C // LANES, scale_row, 0)
            pltpu.sync_copy(rows.at[b], acc.at[cbuf.at[b]], add=True)

        @pl.when(npair > 0)
        def _():
            issue_idx(0, 0)
            issue_idx(1, 1)
            wait_idx(0, 0)
            issue_gather(0)

        def pair(p, carry):
            g = p * 2
            wait_idx(g + 1, 1)
            issue_gather(1)
            process(g, 0)

            @pl.when(p < npair - 1)
            def _():
                issue_idx(g + 2, 0)
                wait_idx(g + 2, 0)
                issue_gather(0)

            process(g + 1, 1)

            @pl.when(p < npair - 1)
            def _():
                issue_idx(g + 3, 1)

            return carry

        lax.fori_loop(0, npair, pair, 0)
        plsc.subcore_barrier()

        @pl.when(cid == 0)
        def _():
            pltpu.sync_copy(acc.at[pl.ds(r0, ROWS_PER_SUB)],
                            out_hbm.at[cid, pl.ds(r0, ROWS_PER_SUB)])

    return agg_kernel(hp, rowp, colp, ewp, zeros_nd)


def _tc_dinv(degp):
    def body(degp_ref, dinv_ref):
        deg = jnp.sum(degp_ref[...], axis=0) + 1.0
        dinv_ref[...] = lax.rsqrt(deg)[:, None]

    return pl.pallas_call(
        body,
        out_shape=jax.ShapeDtypeStruct((N, 1), jnp.float32),
    )(degp)


def _tc_fuse(emb, dl, sl, fw, fb, dinv, w0):
    def body(emb_ref, dl_ref, sl_ref, fw_ref, fb_ref, dinv_ref, w0_ref,
             x0_ref, h0p_ref):
        fw = fw_ref[...]
        x0 = (jnp.dot(emb_ref[...], fw[0:D, :], preferred_element_type=jnp.float32)
              + jnp.dot(dl_ref[...], fw[D:2 * D, :], preferred_element_type=jnp.float32)
              + jnp.dot(sl_ref[...], fw[2 * D:3 * D, :], preferred_element_type=jnp.float32)
              + fb_ref[...])
        x0_ref[...] = x0
        h0 = jnp.dot(x0, w0_ref[...], preferred_element_type=jnp.float32)
        h0p_ref[...] = h0 * dinv_ref[...]

    blk = lambda i: (i, 0)
    return pl.pallas_call(
        body,
        grid=(GRID,),
        in_specs=[
            pl.BlockSpec((RBLK, D), blk),
            pl.BlockSpec((RBLK, D), blk),
            pl.BlockSpec((RBLK, D), blk),
            pl.BlockSpec((3 * D, D), lambda i: (0, 0)),
            pl.BlockSpec((1, D), lambda i: (0, 0)),
            pl.BlockSpec((RBLK, 1), blk),
            pl.BlockSpec((D, D), lambda i: (0, 0)),
        ],
        out_specs=[
            pl.BlockSpec((RBLK, D), blk),
            pl.BlockSpec((RBLK, D), blk),
        ],
        out_shape=[
            jax.ShapeDtypeStruct((N, D), jnp.float32),
            jax.ShapeDtypeStruct((N, D), jnp.float32),
        ],
    )(emb, dl, sl, fw, fb, dinv, w0)


def _tc_mid(aggp, dinv, h0p, x0, b0r, w1):
    def body(aggp_ref, dinv_ref, h0p_ref, x0_ref, b0_ref, w1_ref,
             t1_ref, h1p_ref):
        dinv = dinv_ref[...]
        agg = aggp_ref[0]
        out0 = (agg + h0p_ref[...]) * dinv + b0_ref[...]
        t1 = 0.9 * out0 + 0.1 * x0_ref[...]
        t1_ref[...] = t1
        h1 = jnp.dot(t1, w1_ref[...], preferred_element_type=jnp.float32)
        h1p_ref[...] = h1 * dinv

    blk = lambda i: (i, 0)
    return pl.pallas_call(
        body,
        grid=(GRID,),
        in_specs=[
            pl.BlockSpec((1, RBLK, D), lambda i: (0, i, 0)),
            pl.BlockSpec((RBLK, 1), blk),
            pl.BlockSpec((RBLK, D), blk),
            pl.BlockSpec((RBLK, D), blk),
            pl.BlockSpec((1, D), lambda i: (0, 0)),
            pl.BlockSpec((D, D), lambda i: (0, 0)),
        ],
        out_specs=[
            pl.BlockSpec((RBLK, D), blk),
            pl.BlockSpec((RBLK, D), blk),
        ],
        out_shape=[
            jax.ShapeDtypeStruct((N, D), jnp.float32),
            jax.ShapeDtypeStruct((N, D), jnp.float32),
        ],
    )(aggp, dinv, h0p, x0, b0r, w1)


def _tc_final(aggp, dinv, h1p, t1, b1r):
    def body(aggp_ref, dinv_ref, h1p_ref, t1_ref, b1_ref, out_ref):
        agg = aggp_ref[0]
        out1 = (agg + h1p_ref[...]) * dinv_ref[...] + b1_ref[...]
        out_ref[...] = 0.9 * out1 + 0.1 * t1_ref[...]

    blk = lambda i: (i, 0)
    return pl.pallas_call(
        body,
        grid=(GRID,),
        in_specs=[
            pl.BlockSpec((1, RBLK, D), lambda i: (0, i, 0)),
            pl.BlockSpec((RBLK, 1), blk),
            pl.BlockSpec((RBLK, D), blk),
            pl.BlockSpec((RBLK, D), blk),
            pl.BlockSpec((1, D), lambda i: (0, 0)),
        ],
        out_specs=pl.BlockSpec((RBLK, D), blk),
        out_shape=jax.ShapeDtypeStruct((N, D), jnp.float32),
    )(aggp, dinv, h1p, t1, b1r)


def kernel(demand_seq_emb, supply_seq_emb, l, t_s, t_e, g_d_edge_index,
           g_d_edge_attr, comm, skill_semantic_embed, init_emb, emb_weight,
           fuse_W, fuse_b, W0, b0, W1, b1):
    dlast = demand_seq_emb[:, L_SEQ - 1, :]
    slast = supply_seq_emb[:, L_SEQ - 1, :]

    row = g_d_edge_index[0]
    col = g_d_edge_index[1]
    pad = E_PAD - E
    rowp = jnp.concatenate([row, jnp.zeros((pad,), row.dtype)])
    colp = jnp.concatenate([col, jnp.zeros((pad,), col.dtype)])
    ewp = jnp.concatenate([g_d_edge_attr, jnp.zeros((pad,), g_d_edge_attr.dtype)])
    zeros_nd = jnp.zeros((ACC_N, D), jnp.float32)

    fb = fuse_b.reshape(1, D)
    b0r = b0.reshape(1, D)
    b1r = b1.reshape(1, D)

    degp = _sc_deg(colp, ewp)
    dinv = _tc_dinv(degp)
    x0, h0p = _tc_fuse(emb_weight, dlast, slast, fuse_W, fb, dinv, W0)
    agg0 = _sc_agg(h0p, rowp, colp, ewp, zeros_nd)
    t1, h1p = _tc_mid(agg0, dinv, h0p, x0, b0r, W1)
    agg1 = _sc_agg(h1p, rowp, colp, ewp, zeros_nd)
    skill_embs = _tc_final(agg1, dinv, h1p, t1, b1r)
    return (emb_weight, skill_embs)


# speedup vs baseline: 1.4086x; 1.4086x over previous
import functools

import jax
import jax.numpy as jnp
from jax import lax
from jax.experimental import pallas as pl
from jax.experimental.pallas import tpu as pltpu
from jax.experimental.pallas import tpu_sc as plsc

N = 10000
D = 128
L_SEQ = 12
E = 320000

NC = 2
NS = 16
NW = NC * NS
LANES = 16

C = 128
EPW0 = 16384
EPW1 = 4096
NCH0 = EPW0 // C
NCH1 = EPW1 // C
E_PAD = NS * (EPW0 + EPW1)
EPW_D = E_PAD // NW
CH = 2048
ACC_N = 10240
ROWS_PER_SUB = ACC_N // NS

RBLK = 1000
GRID = N // RBLK


def _mesh():
    return plsc.VectorSubcoreMesh(core_axis_name="c", subcore_axis_name="s")


def _sc_deg(colp, ewp):
    @functools.partial(
        pl.kernel,
        mesh=_mesh(),
        out_type=jax.ShapeDtypeStruct((NW, N), jnp.float32),
        compiler_params=pltpu.CompilerParams(needs_layout_passes=False),
        scratch_types=[
            pltpu.VMEM((N,), jnp.float32),
            pltpu.VMEM((CH,), jnp.int32),
            pltpu.VMEM((CH,), jnp.float32),
        ],
    )
    def deg_kernel(colp_hbm, ewp_hbm, out_hbm, hist, cbuf, ebuf):
        cid = lax.axis_index("c")
        sid = lax.axis_index("s")
        wid = sid * NC + cid

        def zero_body(i, carry):
            hist[pl.ds(i * LANES, LANES)] = jnp.zeros((LANES,), jnp.float32)
            return carry

        lax.fori_loop(0, N // LANES, zero_body, 0)

        def chunk_body(g, carry):
            base = pl.multiple_of(wid * EPW_D + g * CH, 8)
            pltpu.sync_copy(colp_hbm.at[pl.ds(base, CH)], cbuf)
            pltpu.sync_copy(ewp_hbm.at[pl.ds(base, CH)], ebuf)

            def inner(k, carry2):
                sl = pl.ds(k * LANES, LANES)
                plsc.addupdate_scatter(hist, [cbuf[sl]], ebuf[sl])
                return carry2

            lax.fori_loop(0, CH // LANES, inner, 0)
            return carry

        lax.fori_loop(0, EPW_D // CH, chunk_body, 0)
        pltpu.sync_copy(hist, out_hbm.at[wid])

    return deg_kernel(colp, ewp)


def _sc_agg(hp, rowp, colp, ewp, zeros_nd):
    @functools.partial(
        pl.kernel,
        mesh=_mesh(),
        out_type=jax.ShapeDtypeStruct((NC, ACC_N, D), jnp.float32),
        scratch_types=[
            pltpu.VMEM_SHARED((ACC_N, D), jnp.float32),
            pltpu.VMEM((2, C), jnp.int32),
            pltpu.VMEM((2, C), jnp.int32),
            pltpu.VMEM((2, C), jnp.float32),
            pltpu.VMEM((2, C, D), jnp.float32),
            pltpu.SemaphoreType.DMA((2,)),
            pltpu.SemaphoreType.DMA((2,)),
        ],
    )
    def agg_kernel(hp_hbm, rowp_hbm, colp_hbm, ewp_hbm, z_hbm, out_hbm,
                   acc, rbuf, cbuf, ebuf, rows, isem, gsem):
        cid = lax.axis_index("c")
        sid = lax.axis_index("s")
        r0 = sid * ROWS_PER_SUB
        ebase = lax.select(cid == 0, sid * EPW0, NS * EPW0 + sid * EPW1)
        npair = lax.select(cid == 0, NCH0 // 2, NCH1 // 2)

        def zero_body(k, carry):
            for j in range(D // LANES):
                rows[0, k, pl.ds(j * LANES, LANES)] = jnp.zeros((LANES,),
                                                                jnp.float32)
            return carry

        lax.fori_loop(0, C, zero_body, 0)
        for t in range(ROWS_PER_SUB // C):
            pltpu.sync_copy(rows.at[0], acc.at[pl.ds(r0 + t * C, C)])
        plsc.subcore_barrier()

        def idx_copies(g, b):
            base = pl.multiple_of(ebase + g * C, 8)
            return (
                pltpu.make_async_copy(rowp_hbm.at[pl.ds(base, C)],
                                      rbuf.at[b], isem.at[b]),
                pltpu.make_async_copy(colp_hbm.at[pl.ds(base, C)],
                                      cbuf.at[b], isem.at[b]),
                pltpu.make_async_copy(ewp_hbm.at[pl.ds(base, C)],
                                      ebuf.at[b], isem.at[b]),
            )

        def issue_idx(g, b):
            for d in idx_copies(g, b):
                d.start()

        def wait_idx(g, b):
            for d in idx_copies(g, b):
                d.wait()

        def issue_gather(b):
            pltpu.async_copy(hp_hbm.at[rbuf.at[b]], rows.at[b], gsem.at[b])

        def wait_gather(b):
            pltpu.make_async_copy(hp_hbm.at[rbuf.at[b]], rows.at[b],
                                  gsem.at[b]).wait()

        def process(g, b):
            wait_gather(b)

            def scale_row(k, carry2):
                ev = ebuf[b, pl.ds(k * LANES, LANES)]
                for i in range(LANES):
                    r = k * LANES + i
                    s = ev[i]
                    for j in range(D // LANES):
                        sl = pl.ds(j * LANES, LANES)
                        rows[b, r, sl] = rows[b, r, sl] * s
                return carry2

            lax.fori_loop(0, C // LANES, scale_row, 0)
            pltpu.sync_copy(rows.at[b], acc.at[cbuf.at[b]], add=True)

        issue_idx(0, 0)
        issue_idx(1, 1)
        wait_idx(0, 0)
        issue_gather(0)

        def pair(p, carry):
            g = p * 2
            wait_idx(g + 1, 1)
            issue_gather(1)
            process(g, 0)

            @pl.when(p < npair - 1)
            def _():
                issue_idx(g + 2, 0)
                wait_idx(g + 2, 0)
                issue_gather(0)

            process(g + 1, 1)

            @pl.when(p < npair - 1)
            def _():
                issue_idx(g + 3, 1)

            return carry

        lax.fori_loop(0, npair, pair, 0)
        plsc.subcore_barrier()
        pltpu.sync_copy(acc.at[pl.ds(r0, ROWS_PER_SUB)],
                        out_hbm.at[cid, pl.ds(r0, ROWS_PER_SUB)])

    return agg_kernel(hp, rowp, colp, ewp, zeros_nd)


def _tc_dinv(degp):
    def body(degp_ref, dinv_ref):
        deg = jnp.sum(degp_ref[...], axis=0) + 1.0
        dinv_ref[...] = lax.rsqrt(deg)[:, None]

    return pl.pallas_call(
        body,
        out_shape=jax.ShapeDtypeStruct((N, 1), jnp.float32),
    )(degp)


def _tc_fuse(emb, dl, sl, fw, fb, dinv, w0):
    def body(emb_ref, dl_ref, sl_ref, fw_ref, fb_ref, dinv_ref, w0_ref,
             x0_ref, h0p_ref):
        fw = fw_ref[...]
        x0 = (jnp.dot(emb_ref[...], fw[0:D, :], preferred_element_type=jnp.float32)
              + jnp.dot(dl_ref[...], fw[D:2 * D, :], preferred_element_type=jnp.float32)
              + jnp.dot(sl_ref[...], fw[2 * D:3 * D, :], preferred_element_type=jnp.float32)
              + fb_ref[...])
        x0_ref[...] = x0
        h0 = jnp.dot(x0, w0_ref[...], preferred_element_type=jnp.float32)
        h0p_ref[...] = h0 * dinv_ref[...]

    blk = lambda i: (i, 0)
    return pl.pallas_call(
        body,
        grid=(GRID,),
        in_specs=[
            pl.BlockSpec((RBLK, D), blk),
            pl.BlockSpec((RBLK, D), blk),
            pl.BlockSpec((RBLK, D), blk),
            pl.BlockSpec((3 * D, D), lambda i: (0, 0)),
            pl.BlockSpec((1, D), lambda i: (0, 0)),
            pl.BlockSpec((RBLK, 1), blk),
            pl.BlockSpec((D, D), lambda i: (0, 0)),
        ],
        out_specs=[
            pl.BlockSpec((RBLK, D), blk),
            pl.BlockSpec((RBLK, D), blk),
        ],
        out_shape=[
            jax.ShapeDtypeStruct((N, D), jnp.float32),
            jax.ShapeDtypeStruct((N, D), jnp.float32),
        ],
    )(emb, dl, sl, fw, fb, dinv, w0)


def _tc_mid(aggp, dinv, h0p, x0, b0r, w1):
    def body(aggp_ref, dinv_ref, h0p_ref, x0_ref, b0_ref, w1_ref,
             t1_ref, h1p_ref):
        dinv = dinv_ref[...]
        agg = aggp_ref[0] + aggp_ref[1]
        out0 = (agg + h0p_ref[...]) * dinv + b0_ref[...]
        t1 = 0.9 * out0 + 0.1 * x0_ref[...]
        t1_ref[...] = t1
        h1 = jnp.dot(t1, w1_ref[...], preferred_element_type=jnp.float32)
        h1p_ref[...] = h1 * dinv

    blk = lambda i: (i, 0)
    return pl.pallas_call(
        body,
        grid=(GRID,),
        in_specs=[
            pl.BlockSpec((NC, RBLK, D), lambda i: (0, i, 0)),
            pl.BlockSpec((RBLK, 1), blk),
            pl.BlockSpec((RBLK, D), blk),
            pl.BlockSpec((RBLK, D), blk),
            pl.BlockSpec((1, D), lambda i: (0, 0)),
            pl.BlockSpec((D, D), lambda i: (0, 0)),
        ],
        out_specs=[
            pl.BlockSpec((RBLK, D), blk),
            pl.BlockSpec((RBLK, D), blk),
        ],
        out_shape=[
            jax.ShapeDtypeStruct((N, D), jnp.float32),
            jax.ShapeDtypeStruct((N, D), jnp.float32),
        ],
    )(aggp, dinv, h0p, x0, b0r, w1)


def _tc_final(aggp, dinv, h1p, t1, b1r):
    def body(aggp_ref, dinv_ref, h1p_ref, t1_ref, b1_ref, out_ref):
        agg = aggp_ref[0] + aggp_ref[1]
        out1 = (agg + h1p_ref[...]) * dinv_ref[...] + b1_ref[...]
        out_ref[...] = 0.9 * out1 + 0.1 * t1_ref[...]

    blk = lambda i: (i, 0)
    return pl.pallas_call(
        body,
        grid=(GRID,),
        in_specs=[
            pl.BlockSpec((NC, RBLK, D), lambda i: (0, i, 0)),
            pl.BlockSpec((RBLK, 1), blk),
            pl.BlockSpec((RBLK, D), blk),
            pl.BlockSpec((RBLK, D), blk),
            pl.BlockSpec((1, D), lambda i: (0, 0)),
        ],
        out_specs=pl.BlockSpec((RBLK, D), blk),
        out_shape=jax.ShapeDtypeStruct((N, D), jnp.float32),
    )(aggp, dinv, h1p, t1, b1r)


def kernel(demand_seq_emb, supply_seq_emb, l, t_s, t_e, g_d_edge_index,
           g_d_edge_attr, comm, skill_semantic_embed, init_emb, emb_weight,
           fuse_W, fuse_b, W0, b0, W1, b1):
    dlast = demand_seq_emb[:, L_SEQ - 1, :]
    slast = supply_seq_emb[:, L_SEQ - 1, :]

    row = g_d_edge_index[0]
    col = g_d_edge_index[1]
    pad = E_PAD - E
    rowp = jnp.concatenate([row, jnp.zeros((pad,), row.dtype)])
    colp = jnp.concatenate([col, jnp.zeros((pad,), col.dtype)])
    ewp = jnp.concatenate([g_d_edge_attr, jnp.zeros((pad,), g_d_edge_attr.dtype)])
    zeros_nd = jnp.zeros((ACC_N, D), jnp.float32)

    fb = fuse_b.reshape(1, D)
    b0r = b0.reshape(1, D)
    b1r = b1.reshape(1, D)

    degp = _sc_deg(colp, ewp)
    dinv = _tc_dinv(degp)
    x0, h0p = _tc_fuse(emb_weight, dlast, slast, fuse_W, fb, dinv, W0)
    agg0 = _sc_agg(h0p, rowp, colp, ewp, zeros_nd)
    t1, h1p = _tc_mid(agg0, dinv, h0p, x0, b0r, W1)
    agg1 = _sc_agg(h1p, rowp, colp, ewp, zeros_nd)
    skill_embs = _tc_final(agg1, dinv, h1p, t1, b1r)
    return (emb_weight, skill_embs)
